# Initial kernel scaffold; baseline (speedup 1.0000x reference)
import jax
import jax.numpy as jnp
from jax.experimental import pallas as pl


def kernel(hidden_states, Wg, We_gate, We_up, We_down, Ws_gate, Ws_up, Ws_down, Wsg):
    raise NotImplementedError("write your pallas kernel here")


# speedup vs baseline: 2.0574x; 2.0574x over previous
import functools

import jax
import jax.numpy as jnp
from jax import lax
from jax.experimental import pallas as pl
from jax.experimental.pallas import tpu as pltpu

T = 2048
D = 1024
E = 8
K = 2
FF = 768
SFF = 2048


def _router_body(x_ref, wg_ref, w_all_ref):
    logits = lax.dot_general(wg_ref[...], x_ref[...], (((1,), (1,)), ((), ())),
                             preferred_element_type=jnp.float32)
    m = jnp.max(logits, axis=0, keepdims=True)
    ex = jnp.exp(logits - m)
    probs = ex / jnp.sum(ex, axis=0, keepdims=True)
    eids = lax.broadcasted_iota(jnp.int32, (E, T), 0)
    m1 = jnp.max(probs, axis=0, keepdims=True)
    i1 = jnp.min(jnp.where(probs == m1, eids, E), axis=0, keepdims=True)
    mask1 = eids == i1
    probs2 = jnp.where(mask1, -1.0, probs)
    m2 = jnp.max(probs2, axis=0, keepdims=True)
    i2 = jnp.min(jnp.where(probs2 == m2, eids, E), axis=0, keepdims=True)
    mask2 = eids == i2
    s = m1 + m2
    w_all = jnp.where(mask1, m1 / s, 0.0) + jnp.where(mask2, m2 / s, 0.0)
    w_all_ref[...] = w_all[:, None, :]


def _router(x, Wg):
    return pl.pallas_call(
        _router_body,
        out_shape=jax.ShapeDtypeStruct((E, 1, T), jnp.float32),
    )(x, Wg)


def _moe_body(x_ref, wgte_ref, wup_ref, wdn_ref, wall_ref, out_ref):
    e = pl.program_id(0)
    x = x_ref[...]
    g = lax.dot_general(x, wgte_ref[0], (((1,), (1,)), ((), ())),
                        preferred_element_type=jnp.float32)
    u = lax.dot_general(x, wup_ref[0], (((1,), (1,)), ((), ())),
                        preferred_element_type=jnp.float32)
    a = g * jax.nn.sigmoid(g) * u
    h = lax.dot_general(a, wdn_ref[0], (((1,), (1,)), ((), ())),
                        preferred_element_type=jnp.float32)
    w_col = jnp.transpose(wall_ref[0], (1, 0))
    contrib = w_col * h

    @pl.when(e == 0)
    def _init():
        out_ref[...] = contrib

    @pl.when(e != 0)
    def _acc():
        out_ref[...] = out_ref[...] + contrib


def _moe_dense(x, We_gate, We_up, We_down, w_all):
    return pl.pallas_call(
        _moe_body,
        grid=(E,),
        in_specs=[
            pl.BlockSpec((T, D), lambda e: (0, 0)),
            pl.BlockSpec((1, FF, D), lambda e: (e, 0, 0)),
            pl.BlockSpec((1, FF, D), lambda e: (e, 0, 0)),
            pl.BlockSpec((1, D, FF), lambda e: (e, 0, 0)),
            pl.BlockSpec((1, 1, T), lambda e: (e, 0, 0)),
        ],
        out_specs=pl.BlockSpec((T, D), lambda e: (0, 0)),
        out_shape=jax.ShapeDtypeStruct((T, D), jnp.float32),
    )(x, We_gate, We_up, We_down, w_all)


def _shared_body(x_ref, routed_ref, wsg_ref, wsu_ref, wsd_ref, wgate_ref, out_ref):
    x = x_ref[...]
    g = lax.dot_general(x, wsg_ref[...], (((1,), (1,)), ((), ())),
                        preferred_element_type=jnp.float32)
    u = lax.dot_general(x, wsu_ref[...], (((1,), (1,)), ((), ())),
                        preferred_element_type=jnp.float32)
    a = g * jax.nn.sigmoid(g) * u
    sh = lax.dot_general(a, wsd_ref[...], (((1,), (1,)), ((), ())),
                         preferred_element_type=jnp.float32)
    gate = jax.nn.sigmoid(
        lax.dot_general(x, wgate_ref[...], (((1,), (1,)), ((), ())),
                        preferred_element_type=jnp.float32))
    out_ref[...] = routed_ref[...] + gate * sh


def _shared(x, routed, Ws_gate, Ws_up, Ws_down, Wsg, bt=256):
    return pl.pallas_call(
        _shared_body,
        grid=(T // bt,),
        in_specs=[
            pl.BlockSpec((bt, D), lambda t: (t, 0)),
            pl.BlockSpec((bt, D), lambda t: (t, 0)),
            pl.BlockSpec((SFF, D), lambda t: (0, 0)),
            pl.BlockSpec((SFF, D), lambda t: (0, 0)),
            pl.BlockSpec((D, SFF), lambda t: (0, 0)),
            pl.BlockSpec((1, D), lambda t: (0, 0)),
        ],
        out_specs=pl.BlockSpec((bt, D), lambda t: (t, 0)),
        out_shape=jax.ShapeDtypeStruct((T, D), jnp.float32),
    )(x, routed, Ws_gate, Ws_up, Ws_down, Wsg)


def kernel(hidden_states, Wg, We_gate, We_up, We_down, Ws_gate, Ws_up, Ws_down, Wsg):
    x = hidden_states.reshape(-1, hidden_states.shape[-1])
    w_all = _router(x, Wg)
    routed = _moe_dense(x, We_gate, We_up, We_down, w_all)
    out = _shared(x, routed, Ws_gate, Ws_up, Ws_down, Wsg)
    return out.reshape(hidden_states.shape)
